# Initial kernel scaffold; baseline (speedup 1.0000x reference)
import functools

import jax
import jax.numpy as jnp
from jax import lax
from jax.experimental import pallas as pl
from jax.experimental.pallas import tpu as pltpu
from jax.experimental.pallas import tpu_sc as plsc

N = 10000
E = 320000
D = 128

NC = 2
NS = 16
NW = NC * NS
EPT = E // NW
K = 80
CHUNKS = EPT // K
RPT = N // NS
ZR = 25
CW = 16

_mesh = plsc.VectorSubcoreMesh(core_axis_name="c", subcore_axis_name="s")


@functools.partial(
    pl.kernel,
    mesh=_mesh,
    out_type=[
        jax.ShapeDtypeStruct((NC * N, D), jnp.float32),
        jax.ShapeDtypeStruct((NC * N, CW), jnp.float32),
    ],
    scratch_types=[
        pltpu.VMEM((K,), jnp.int32),
        pltpu.VMEM((K,), jnp.int32),
        pltpu.VMEM((K, D), jnp.float32),
        pltpu.VMEM((K, CW), jnp.float32),
        pltpu.VMEM((ZR, D), jnp.float32),
        pltpu.VMEM((ZR, CW), jnp.float32),
        pltpu.VMEM_SHARED((N, D), jnp.float32),
        pltpu.VMEM_SHARED((N, CW), jnp.float32),
        pltpu.SemaphoreType.DMA,
    ],
)
def _sc_aggregate(table_hbm, src_hbm, dst_hbm, acc_out, cnt_out,
                  src_v, dst_v, rows_v, ones_v, zrow_v, zcnt_v,
                  acc_sh, cnt_sh, sem):
    cid = lax.axis_index("c")
    sid = lax.axis_index("s")
    wid = cid * NS + sid

    zero16 = jnp.zeros((16,), jnp.float32)
    one16 = jnp.ones((16,), jnp.float32)
    for r in range(ZR):
        for c8 in range(D // 16):
            zrow_v[r, pl.ds(c8 * 16, 16)] = zero16
        zcnt_v[r, :] = zero16
    for r in range(K):
        ones_v[r, :] = one16

    for j in range(RPT // ZR):
        r0 = sid * RPT + j * ZR
        pltpu.sync_copy(zrow_v, acc_sh.at[pl.ds(r0, ZR)])
        pltpu.sync_copy(zcnt_v, cnt_sh.at[pl.ds(r0, ZR)])
    plsc.subcore_barrier()

    def body(i, carry):
        base = wid * EPT + i * K
        pltpu.sync_copy(src_hbm.at[pl.ds(base, K)], src_v)
        pltpu.sync_copy(dst_hbm.at[pl.ds(base, K)], dst_v)
        pltpu.async_copy(table_hbm.at[src_v], rows_v, sem).wait()
        pltpu.sync_copy(rows_v, acc_sh.at[dst_v], add=True)
        pltpu.sync_copy(ones_v, cnt_sh.at[dst_v], add=True)
        return carry

    lax.fori_loop(0, CHUNKS, body, 0)
    plsc.subcore_barrier()

    for j in range(RPT // ZR):
        r0 = sid * RPT + j * ZR
        pltpu.sync_copy(acc_sh.at[pl.ds(r0, ZR)], zrow_v)
        pltpu.sync_copy(zrow_v, acc_out.at[pl.ds(cid * N + r0, ZR)])
        pltpu.sync_copy(cnt_sh.at[pl.ds(r0, ZR)], zcnt_v)
        pltpu.sync_copy(zcnt_v, cnt_out.at[pl.ds(cid * N + r0, ZR)])


_RB = 1000


def _dense_body(relu, acc0_ref, acc1_ref, cnt0_ref, cnt1_ref, xin_ref,
                wl_ref, bl_ref, wr_ref, br_ref, out_ref):
    cnt = cnt0_ref[:, 0:1] + cnt1_ref[:, 0:1]
    mean = (acc0_ref[...] + acc1_ref[...]) / jnp.maximum(cnt, 1.0)
    z = lax.dot_general(mean, wl_ref[...], (((1,), (1,)), ((), ())),
                        preferred_element_type=jnp.float32)
    z = z + lax.dot_general(xin_ref[...], wr_ref[...], (((1,), (1,)), ((), ())),
                            preferred_element_type=jnp.float32)
    z = z + bl_ref[...] + br_ref[...]
    if relu:
        out_ref[...] = jnp.maximum(z, 0.0)
    else:
        m = jnp.max(z, axis=-1, keepdims=True)
        e = z - m
        out_ref[...] = e - jnp.log(jnp.sum(jnp.exp(e), axis=-1, keepdims=True))


_RB = 1000


def _dense(relu, acc0, acc1, cnt0, cnt1, xin, wl, bl, wr, br):
    grid = (N // _RB,)
    row_spec = pl.BlockSpec((_RB, D), lambda i: (i, 0))
    cnt_spec = pl.BlockSpec((_RB, CW), lambda i: (i, 0))
    w_spec = pl.BlockSpec((D, D), lambda i: (0, 0))
    b_spec = pl.BlockSpec((1, D), lambda i: (0, 0))
    return pl.pallas_call(
        functools.partial(_dense_body, relu),
        grid=grid,
        in_specs=[row_spec, row_spec, cnt_spec, cnt_spec, row_spec,
                  w_spec, b_spec, w_spec, b_spec],
        out_specs=row_spec,
        out_shape=jax.ShapeDtypeStruct((N, D), jnp.float32),
    )(acc0, acc1, cnt0, cnt1, xin, wl, bl, wr, br)


def kernel(x, edge_index, Wl1, bl1, Wr1, br1, Wl2, bl2, Wr2, br2):
    src = edge_index[0]
    dst = edge_index[1]
    bl1 = bl1.reshape(1, D)
    br1 = br1.reshape(1, D)
    bl2 = bl2.reshape(1, D)
    br2 = br2.reshape(1, D)

    acc1p, cnt1p = _sc_aggregate(x, src, dst)
    h = _dense(True, acc1p[:N], acc1p[N:], cnt1p[:N], cnt1p[N:],
               x, Wl1, bl1, Wr1, br1)
    acc2p, cnt2p = _sc_aggregate(h, src, dst)
    out = _dense(False, acc2p[:N], acc2p[N:], cnt2p[:N], cnt2p[N:],
                 h, Wl2, bl2, Wr2, br2)
    return out


# speedup vs baseline: 4.6774x; 4.6774x over previous
import functools

import jax
import jax.numpy as jnp
from jax import lax
from jax.experimental import pallas as pl
from jax.experimental.pallas import tpu as pltpu
from jax.experimental.pallas import tpu_sc as plsc

N = 10000
E = 320000
D = 128

NC = 2
NS = 16
NW = NC * NS
EPT = E // NW
K = 80
CHUNKS = EPT // K
NP = 10240
RPT = NP // NS

_mesh = plsc.VectorSubcoreMesh(core_axis_name="c", subcore_axis_name="s")


@functools.partial(
    pl.kernel,
    mesh=_mesh,
    out_type=jax.ShapeDtypeStruct((NC * NP, D), jnp.float32),
    scratch_types=[
        pltpu.VMEM((K,), jnp.int32),
        pltpu.VMEM((K,), jnp.int32),
        pltpu.VMEM((K, D), jnp.float32),
        pltpu.VMEM_SHARED((NP, D), jnp.float32),
        pltpu.SemaphoreType.DMA,
    ],
)
def _sc_aggregate(table_hbm, src_hbm, dst_hbm, zrow_hbm,
                  acc_out, src_v, dst_v, rows_v, acc_sh, sem):
    cid = lax.axis_index("c")
    sid = lax.axis_index("s")
    wid = cid * NS + sid
    r0 = sid * RPT

    pltpu.sync_copy(zrow_hbm.at[pl.ds(r0, RPT)], acc_sh.at[pl.ds(r0, RPT)])
    plsc.subcore_barrier()

    def body(i, carry):
        base = wid * EPT + i * K
        pltpu.sync_copy(src_hbm.at[pl.ds(base, K)], src_v)
        pltpu.sync_copy(dst_hbm.at[pl.ds(base, K)], dst_v)
        pltpu.async_copy(table_hbm.at[src_v], rows_v, sem).wait()
        pltpu.sync_copy(rows_v, acc_sh.at[dst_v], add=True)
        return carry

    lax.fori_loop(0, CHUNKS, body, 0)
    plsc.subcore_barrier()

    pltpu.sync_copy(acc_sh.at[pl.ds(r0, RPT)], acc_out.at[pl.ds(cid * NP + r0, RPT)])


@functools.partial(
    pl.kernel,
    mesh=_mesh,
    out_type=jax.ShapeDtypeStruct((NC * NP, D), jnp.float32),
    scratch_types=[
        pltpu.VMEM((K,), jnp.int32),
        pltpu.VMEM((K, D), jnp.float32),
        pltpu.VMEM_SHARED((NP, D), jnp.float32),
    ],
)
def _sc_counts(dst_hbm, zrow_hbm, cnt_out, dst_v, ones_v, cnt_sh):
    cid = lax.axis_index("c")
    sid = lax.axis_index("s")
    wid = cid * NS + sid
    r0 = sid * RPT

    one16 = jnp.ones((16,), jnp.float32)
    for r in range(K):
        for c8 in range(D // 16):
            ones_v[r, pl.ds(c8 * 16, 16)] = one16

    pltpu.sync_copy(zrow_hbm.at[pl.ds(r0, RPT)], cnt_sh.at[pl.ds(r0, RPT)])
    plsc.subcore_barrier()

    def body(i, carry):
        base = wid * EPT + i * K
        pltpu.sync_copy(dst_hbm.at[pl.ds(base, K)], dst_v)
        pltpu.sync_copy(ones_v, cnt_sh.at[dst_v], add=True)
        return carry

    lax.fori_loop(0, CHUNKS, body, 0)
    plsc.subcore_barrier()

    pltpu.sync_copy(cnt_sh.at[pl.ds(r0, RPT)], cnt_out.at[pl.ds(cid * NP + r0, RPT)])


_RB = 1000


def _dense_body(relu, acc0_ref, acc1_ref, cnt0_ref, cnt1_ref, xin_ref,
                wl_ref, bl_ref, wr_ref, br_ref, out_ref):
    cnt = cnt0_ref[:, 0:1] + cnt1_ref[:, 0:1]
    mean = (acc0_ref[...] + acc1_ref[...]) / jnp.maximum(cnt, 1.0)
    z = lax.dot_general(mean, wl_ref[...], (((1,), (1,)), ((), ())),
                        preferred_element_type=jnp.float32)
    z = z + lax.dot_general(xin_ref[...], wr_ref[...], (((1,), (1,)), ((), ())),
                            preferred_element_type=jnp.float32)
    z = z + bl_ref[...] + br_ref[...]
    if relu:
        out_ref[...] = jnp.maximum(z, 0.0)
    else:
        m = jnp.max(z, axis=-1, keepdims=True)
        e = z - m
        out_ref[...] = e - jnp.log(jnp.sum(jnp.exp(e), axis=-1, keepdims=True))


def _dense(relu, acc0, acc1, cnt0, cnt1, xin, wl, bl, wr, br):
    grid = (N // _RB,)
    row_spec = pl.BlockSpec((_RB, D), lambda i: (i, 0))
    w_spec = pl.BlockSpec((D, D), lambda i: (0, 0))
    b_spec = pl.BlockSpec((1, D), lambda i: (0, 0))
    return pl.pallas_call(
        functools.partial(_dense_body, relu),
        grid=grid,
        in_specs=[row_spec, row_spec, row_spec, row_spec, row_spec,
                  w_spec, b_spec, w_spec, b_spec],
        out_specs=row_spec,
        out_shape=jax.ShapeDtypeStruct((N, D), jnp.float32),
    )(acc0, acc1, cnt0, cnt1, xin, wl, bl, wr, br)


def kernel(x, edge_index, Wl1, bl1, Wr1, br1, Wl2, bl2, Wr2, br2):
    src = edge_index[0]
    dst = edge_index[1]
    bl1 = bl1.reshape(1, D)
    br1 = br1.reshape(1, D)
    bl2 = bl2.reshape(1, D)
    br2 = br2.reshape(1, D)

    zrow = jnp.zeros((NP, D), jnp.float32)
    cntp = _sc_counts(dst, zrow)
    cnt0, cnt1 = cntp[:N], cntp[NP:NP + N]
    acc1p = _sc_aggregate(x, src, dst, zrow)
    h = _dense(True, acc1p[:N], acc1p[NP:NP + N], cnt0, cnt1,
               x, Wl1, bl1, Wr1, br1)
    acc2p = _sc_aggregate(h, src, dst, zrow)
    out = _dense(False, acc2p[:N], acc2p[NP:NP + N], cnt0, cnt1,
                 h, Wl2, bl2, Wr2, br2)
    return out


# speedup vs baseline: 8.5509x; 1.8281x over previous
import functools

import jax
import jax.numpy as jnp
from jax import lax
from jax.experimental import pallas as pl
from jax.experimental.pallas import tpu as pltpu
from jax.experimental.pallas import tpu_sc as plsc

N = 10000
E = 320000
D = 128

NC = 2
NS = 16
NW = NC * NS
EPT = E // NW
K = 125
CHUNKS = EPT // K
SC_CH = 8
SUPERS = CHUNKS // SC_CH
NP = 10240
RPT = NP // NS

_mesh = plsc.VectorSubcoreMesh(core_axis_name="c", subcore_axis_name="s")


@functools.partial(
    pl.kernel,
    mesh=_mesh,
    out_type=jax.ShapeDtypeStruct((NC * NP, D), jnp.float32),
    scratch_types=[
        pltpu.VMEM((SC_CH, K), jnp.int32),
        pltpu.VMEM((SC_CH, K), jnp.int32),
        pltpu.VMEM((K, D), jnp.float32),
        pltpu.VMEM((K, D), jnp.float32),
        pltpu.VMEM_SHARED((NP, D), jnp.float32),
        pltpu.SemaphoreType.DMA,
        pltpu.SemaphoreType.DMA,
    ],
)
def _sc_aggregate(table_hbm, src_hbm, dst_hbm, zrow_hbm,
                  acc_out, src_v, dst_v, rows0_v, rows1_v, acc_sh,
                  sem0, sem1):
    cid = lax.axis_index("c")
    sid = lax.axis_index("s")
    wid = cid * NS + sid
    r0 = sid * RPT

    pltpu.sync_copy(zrow_hbm.at[pl.ds(r0, RPT)], acc_sh.at[pl.ds(r0, RPT)])
    plsc.subcore_barrier()

    def super_body(s, carry):
        base = wid * CHUNKS + s * SC_CH
        pltpu.sync_copy(src_hbm.at[pl.ds(base, SC_CH)], src_v)
        pltpu.sync_copy(dst_hbm.at[pl.ds(base, SC_CH)], dst_v)
        pltpu.async_copy(table_hbm.at[src_v.at[0]], rows0_v, sem0)
        for j in range(SC_CH // 2):
            i0 = 2 * j
            i1 = i0 + 1
            pltpu.make_async_copy(table_hbm.at[src_v.at[i0]], rows0_v, sem0).wait()
            pltpu.async_copy(table_hbm.at[src_v.at[i1]], rows1_v, sem1)
            pltpu.sync_copy(rows0_v, acc_sh.at[dst_v.at[i0]], add=True)
            pltpu.make_async_copy(table_hbm.at[src_v.at[i1]], rows1_v, sem1).wait()
            if i0 + 2 < SC_CH:
                pltpu.async_copy(table_hbm.at[src_v.at[i0 + 2]], rows0_v, sem0)
            pltpu.sync_copy(rows1_v, acc_sh.at[dst_v.at[i1]], add=True)
        return carry

    lax.fori_loop(0, SUPERS, super_body, 0)
    plsc.subcore_barrier()

    pltpu.sync_copy(acc_sh.at[pl.ds(r0, RPT)], acc_out.at[pl.ds(cid * NP + r0, RPT)])


@functools.partial(
    pl.kernel,
    mesh=_mesh,
    out_type=jax.ShapeDtypeStruct((NC * NP, D), jnp.float32),
    scratch_types=[
        pltpu.VMEM((SC_CH, K), jnp.int32),
        pltpu.VMEM((K, D), jnp.float32),
        pltpu.VMEM_SHARED((NP, D), jnp.float32),
    ],
)
def _sc_counts(dst_hbm, zrow_hbm, cnt_out, dst_v, ones_v, cnt_sh):
    cid = lax.axis_index("c")
    sid = lax.axis_index("s")
    wid = cid * NS + sid
    r0 = sid * RPT

    one16 = jnp.ones((16,), jnp.float32)
    for r in range(K):
        for c8 in range(D // 16):
            ones_v[r, pl.ds(c8 * 16, 16)] = one16

    pltpu.sync_copy(zrow_hbm.at[pl.ds(r0, RPT)], cnt_sh.at[pl.ds(r0, RPT)])
    plsc.subcore_barrier()

    def super_body(s, carry):
        base = wid * CHUNKS + s * SC_CH
        pltpu.sync_copy(dst_hbm.at[pl.ds(base, SC_CH)], dst_v)
        for j in range(SC_CH):
            pltpu.sync_copy(ones_v, cnt_sh.at[dst_v.at[j]], add=True)
        return carry

    lax.fori_loop(0, SUPERS, super_body, 0)
    plsc.subcore_barrier()

    pltpu.sync_copy(cnt_sh.at[pl.ds(r0, RPT)], cnt_out.at[pl.ds(cid * NP + r0, RPT)])


_RB = 1000


def _dense_body(relu, acc0_ref, acc1_ref, cnt0_ref, cnt1_ref, xin_ref,
                wl_ref, bl_ref, wr_ref, br_ref, out_ref):
    cnt = cnt0_ref[:, 0:1] + cnt1_ref[:, 0:1]
    mean = (acc0_ref[...] + acc1_ref[...]) / jnp.maximum(cnt, 1.0)
    z = lax.dot_general(mean, wl_ref[...], (((1,), (1,)), ((), ())),
                        preferred_element_type=jnp.float32)
    z = z + lax.dot_general(xin_ref[...], wr_ref[...], (((1,), (1,)), ((), ())),
                            preferred_element_type=jnp.float32)
    z = z + bl_ref[...] + br_ref[...]
    if relu:
        out_ref[...] = jnp.maximum(z, 0.0)
    else:
        m = jnp.max(z, axis=-1, keepdims=True)
        e = z - m
        out_ref[...] = e - jnp.log(jnp.sum(jnp.exp(e), axis=-1, keepdims=True))


def _dense(relu, acc0, acc1, cnt0, cnt1, xin, wl, bl, wr, br):
    grid = (N // _RB,)
    row_spec = pl.BlockSpec((_RB, D), lambda i: (i, 0))
    w_spec = pl.BlockSpec((D, D), lambda i: (0, 0))
    b_spec = pl.BlockSpec((1, D), lambda i: (0, 0))
    return pl.pallas_call(
        functools.partial(_dense_body, relu),
        grid=grid,
        in_specs=[row_spec, row_spec, row_spec, row_spec, row_spec,
                  w_spec, b_spec, w_spec, b_spec],
        out_specs=row_spec,
        out_shape=jax.ShapeDtypeStruct((N, D), jnp.float32),
    )(acc0, acc1, cnt0, cnt1, xin, wl, bl, wr, br)


def kernel(x, edge_index, Wl1, bl1, Wr1, br1, Wl2, bl2, Wr2, br2):
    src = edge_index[0].reshape(NW * CHUNKS, K)
    dst = edge_index[1].reshape(NW * CHUNKS, K)
    bl1 = bl1.reshape(1, D)
    br1 = br1.reshape(1, D)
    bl2 = bl2.reshape(1, D)
    br2 = br2.reshape(1, D)

    zrow = jnp.zeros((NP, D), jnp.float32)
    cntp = _sc_counts(dst, zrow)
    cnt0, cnt1 = cntp[:N], cntp[NP:NP + N]
    acc1p = _sc_aggregate(x, src, dst, zrow)
    h = _dense(True, acc1p[:N], acc1p[NP:NP + N], cnt0, cnt1,
               x, Wl1, bl1, Wr1, br1)
    acc2p = _sc_aggregate(h, src, dst, zrow)
    out = _dense(False, acc2p[:N], acc2p[NP:NP + N], cnt0, cnt1,
                 h, Wl2, bl2, Wr2, br2)
    return out
